# Initial kernel scaffold; baseline (speedup 1.0000x reference)
import functools

import jax
import jax.numpy as jnp
from jax import lax
from jax.experimental import pallas as pl
from jax.experimental.pallas import tpu as pltpu
from jax.experimental.pallas import tpu_sc as plsc

_BATCH = 16384
_SEQ = 200
_KMERS = 1024
_TOTAL = _BATCH * _SEQ
_NW = 32
_PER_W = _TOTAL // _NW
_CHUNK = 12800
_NCHUNK = _PER_W // _CHUNK
_LANES = 16


def _gather_kernel(idx_hbm, table_hbm, out_hbm, tab_v, idx_v, out_v):
    wid = lax.axis_index("s") * 2 + lax.axis_index("c")
    base = wid * _PER_W

    pltpu.sync_copy(table_hbm, tab_v)

    def _exp_body(i):
        sl = pl.ds(i * _LANES, _LANES)
        tab_v[sl] = jnp.exp(tab_v[sl])

    pl.loop(0, _KMERS // _LANES)(_exp_body)

    def _chunk_body(c):
        off = base + c * _CHUNK
        pltpu.sync_copy(idx_hbm.at[pl.ds(off, _CHUNK)], idx_v)

        def _group_body(g):
            sl = pl.ds(g * _LANES, _LANES)
            out_v[sl] = plsc.load_gather(tab_v, [idx_v[sl]])

        pl.loop(0, _CHUNK // _LANES, unroll=8)(_group_body)

        pltpu.sync_copy(out_v, out_hbm.at[pl.ds(off, _CHUNK)])

    pl.loop(0, _NCHUNK)(_chunk_body)


@jax.jit
def kernel(encoded_parents, masks, kmer_embedding):
    del masks
    idx_flat = encoded_parents.reshape(_TOTAL)
    table_flat = kmer_embedding.reshape(_KMERS)

    mesh = plsc.VectorSubcoreMesh(core_axis_name="c", subcore_axis_name="s")
    out = pl.kernel(
        _gather_kernel,
        mesh=mesh,
        out_type=jax.ShapeDtypeStruct((_TOTAL,), jnp.float32),
        scratch_types=[
            pltpu.VMEM((_KMERS,), jnp.float32),
            pltpu.VMEM((_CHUNK,), jnp.int32),
            pltpu.VMEM((_CHUNK,), jnp.float32),
        ],
    )(idx_flat, table_flat)
    return out.reshape(_BATCH, _SEQ)


# speedup vs baseline: 154.5391x; 154.5391x over previous
import functools

import jax
import jax.numpy as jnp
from jax import lax
from jax.experimental import pallas as pl
from jax.experimental.pallas import tpu as pltpu
from jax.experimental.pallas import tpu_sc as plsc

_BATCH = 16384
_SEQ = 200
_KMERS = 1024
_TOTAL = _BATCH * _SEQ
_NW = 32
_PER_W = _TOTAL // _NW
_CHUNK = 12800
_NCHUNK = _PER_W // _CHUNK
_LANES = 16


def _gather_kernel(idx_hbm, table_hbm, out_hbm, tab_v, idx_v, out_v):
    wid = lax.axis_index("s") * 2 + lax.axis_index("c")
    base = wid * _PER_W

    pltpu.sync_copy(table_hbm, tab_v)

    def _exp_body(i):
        sl = pl.ds(i * _LANES, _LANES)
        tab_v[sl] = jnp.exp(tab_v[sl])

    pl.loop(0, _KMERS // _LANES)(_exp_body)

    def _chunk_body(c):
        off = base + c * _CHUNK
        pltpu.sync_copy(idx_hbm.at[pl.ds(off, _CHUNK)], idx_v)

        def _group_body(g):
            sl = pl.ds(g * _LANES, _LANES)
            out_v[sl] = plsc.load_gather(tab_v, [idx_v[sl]])

        pl.loop(0, _CHUNK // _LANES, unroll=8)(_group_body)

        pltpu.sync_copy(out_v, out_hbm.at[pl.ds(off, _CHUNK)])

    pl.loop(0, _NCHUNK)(_chunk_body)


@jax.jit
def kernel(encoded_parents, masks, kmer_embedding):
    del masks
    idx_flat = encoded_parents.reshape(_TOTAL)
    table_flat = kmer_embedding.reshape(_KMERS)

    mesh = plsc.VectorSubcoreMesh(core_axis_name="c", subcore_axis_name="s")
    out = pl.kernel(
        _gather_kernel,
        mesh=mesh,
        out_type=jax.ShapeDtypeStruct((_TOTAL,), jnp.float32),
        scratch_types=[
            pltpu.VMEM((_KMERS,), jnp.float32),
            pltpu.VMEM((_CHUNK,), jnp.int32),
            pltpu.VMEM((_CHUNK,), jnp.float32),
        ],
        compiler_params=pltpu.CompilerParams(needs_layout_passes=False),
    )(idx_flat, table_flat)
    return out.reshape(_BATCH, _SEQ)


# speedup vs baseline: 233.4328x; 1.5105x over previous
import jax
import jax.numpy as jnp
from jax import lax
from jax.experimental import pallas as pl
from jax.experimental.pallas import tpu as pltpu
from jax.experimental.pallas import tpu_sc as plsc

_BATCH = 16384
_SEQ = 200
_KMERS = 1024
_TOTAL = _BATCH * _SEQ
_NW = 32
_PER_W = _TOTAL // _NW
_CHUNK = 12800
_NCHUNK = _PER_W // _CHUNK
_NBUF = 2
_LANES = 16


def _gather_kernel(idx_hbm, table_hbm, out_hbm, tab_v, idx_v0, idx_v1,
                   out_v0, out_v1, in_sems, out_sems):
    idx_bufs = (idx_v0, idx_v1)
    out_bufs = (out_v0, out_v1)
    wid = lax.axis_index("s") * 2 + lax.axis_index("c")
    base = wid * _PER_W

    pltpu.sync_copy(table_hbm, tab_v)

    def _exp_body(i):
        sl = pl.ds(i * _LANES, _LANES)
        tab_v[sl] = jnp.exp(tab_v[sl])

    pl.loop(0, _KMERS // _LANES)(_exp_body)

    def _in_slice(c):
        return idx_hbm.at[pl.ds(base + c * _CHUNK, _CHUNK)]

    def _out_slice(c):
        return out_hbm.at[pl.ds(base + c * _CHUNK, _CHUNK)]

    for b in range(_NBUF):
        pltpu.async_copy(_in_slice(b), idx_bufs[b], in_sems.at[b])

    for c in range(_NCHUNK):
        b = c % _NBUF
        idx_b = idx_bufs[b]
        out_b = out_bufs[b]
        pltpu.make_async_copy(_in_slice(c), idx_b, in_sems.at[b]).wait()
        if c >= _NBUF:
            pltpu.make_async_copy(out_b, _out_slice(c - _NBUF),
                                  out_sems.at[b]).wait()

        def _group_body(g):
            sl = pl.ds(g * _LANES, _LANES)
            out_b[sl] = plsc.load_gather(tab_v, [idx_b[sl]])

        plsc.parallel_loop(0, _CHUNK // _LANES, unroll=8)(_group_body)

        pltpu.async_copy(out_b, _out_slice(c), out_sems.at[b])
        if c + _NBUF < _NCHUNK:
            pltpu.async_copy(_in_slice(c + _NBUF), idx_b, in_sems.at[b])

    for c in range(_NCHUNK - _NBUF, _NCHUNK):
        b = c % _NBUF
        pltpu.make_async_copy(out_bufs[b], _out_slice(c),
                              out_sems.at[b]).wait()


@jax.jit
def kernel(encoded_parents, masks, kmer_embedding):
    del masks
    idx_flat = encoded_parents.reshape(_TOTAL)
    table_flat = kmer_embedding.reshape(_KMERS)

    mesh = plsc.VectorSubcoreMesh(core_axis_name="c", subcore_axis_name="s")
    out = pl.kernel(
        _gather_kernel,
        mesh=mesh,
        out_type=jax.ShapeDtypeStruct((_TOTAL,), jnp.float32),
        scratch_types=[
            pltpu.VMEM((_KMERS,), jnp.float32),
            pltpu.VMEM((_CHUNK,), jnp.int32),
            pltpu.VMEM((_CHUNK,), jnp.int32),
            pltpu.VMEM((_CHUNK,), jnp.float32),
            pltpu.VMEM((_CHUNK,), jnp.float32),
            pltpu.SemaphoreType.DMA((_NBUF,)),
            pltpu.SemaphoreType.DMA((_NBUF,)),
        ],
        compiler_params=pltpu.CompilerParams(needs_layout_passes=False),
    )(idx_flat, table_flat)
    return out.reshape(_BATCH, _SEQ)


# speedup vs baseline: 411.0323x; 1.7608x over previous
import jax
import jax.numpy as jnp
from jax import lax
from jax.experimental import pallas as pl
from jax.experimental.pallas import tpu as pltpu
from jax.experimental.pallas import tpu_sc as plsc

_BATCH = 16384
_SEQ = 200
_KMERS = 1024
_NW = 32
_ROWS_W = _BATCH // _NW
_CROWS = 64
_NCHUNK = _ROWS_W // _CROWS
_NBUF = 2
_LANES = 16
_NGROUPS = 13


def _gather_kernel(idx_hbm, table_hbm, out_hbm, tab_v, idx_v0, idx_v1,
                   out_v0, out_v1, in_sems, out_sems):
    idx_bufs = (idx_v0, idx_v1)
    out_bufs = (out_v0, out_v1)
    wid = lax.axis_index("s") * 2 + lax.axis_index("c")
    base = wid * _ROWS_W

    pltpu.sync_copy(table_hbm, tab_v)

    def _exp_body(i):
        sl = pl.ds(i * _LANES, _LANES)
        tab_v[sl] = jnp.exp(tab_v[sl])

    pl.loop(0, _KMERS // _LANES)(_exp_body)

    def _in_slice(c):
        return idx_hbm.at[pl.ds(base + c * _CROWS, _CROWS)]

    def _out_slice(c):
        return out_hbm.at[pl.ds(base + c * _CROWS, _CROWS)]

    for b in range(_NBUF):
        pltpu.async_copy(_in_slice(b), idx_bufs[b], in_sems.at[b])

    for c in range(_NCHUNK):
        b = c % _NBUF
        idx_b = idx_bufs[b]
        out_b = out_bufs[b]
        pltpu.make_async_copy(_in_slice(c), idx_b, in_sems.at[b]).wait()
        if c >= _NBUF:
            pltpu.make_async_copy(out_b, _out_slice(c - _NBUF),
                                  out_sems.at[b]).wait()

        def _row_body(r):
            for g in range(_NGROUPS):
                col = min(g * _LANES, _SEQ - _LANES)
                sl = pl.ds(col, _LANES)
                out_b[r, sl] = plsc.load_gather(tab_v, [idx_b[r, sl]])

        plsc.parallel_loop(0, _CROWS)(_row_body)

        pltpu.async_copy(out_b, _out_slice(c), out_sems.at[b])
        if c + _NBUF < _NCHUNK:
            pltpu.async_copy(_in_slice(c + _NBUF), idx_b, in_sems.at[b])

    for c in range(_NCHUNK - _NBUF, _NCHUNK):
        b = c % _NBUF
        pltpu.make_async_copy(out_bufs[b], _out_slice(c),
                              out_sems.at[b]).wait()


@jax.jit
def kernel(encoded_parents, masks, kmer_embedding):
    del masks
    table_flat = kmer_embedding.reshape(_KMERS)

    mesh = plsc.VectorSubcoreMesh(core_axis_name="c", subcore_axis_name="s")
    out = pl.kernel(
        _gather_kernel,
        mesh=mesh,
        out_type=jax.ShapeDtypeStruct((_BATCH, _SEQ), jnp.float32),
        scratch_types=[
            pltpu.VMEM((_KMERS,), jnp.float32),
            pltpu.VMEM((_CROWS, _SEQ), jnp.int32),
            pltpu.VMEM((_CROWS, _SEQ), jnp.int32),
            pltpu.VMEM((_CROWS, _SEQ), jnp.float32),
            pltpu.VMEM((_CROWS, _SEQ), jnp.float32),
            pltpu.SemaphoreType.DMA((_NBUF,)),
            pltpu.SemaphoreType.DMA((_NBUF,)),
        ],
        compiler_params=pltpu.CompilerParams(needs_layout_passes=False),
    )(encoded_parents, table_flat)
    return out


# speedup vs baseline: 411.4490x; 1.0010x over previous
import jax
import jax.numpy as jnp
from jax import lax
from jax.experimental import pallas as pl
from jax.experimental.pallas import tpu as pltpu
from jax.experimental.pallas import tpu_sc as plsc

_BATCH = 16384
_SEQ = 200
_KMERS = 1024
_NW = 32
_ROWS_W = _BATCH // _NW
_CROWS = 64
_NCHUNK = _ROWS_W // _CROWS
_NBUF = 2
_LANES = 16
_NGROUPS = 13


def _gather_kernel(idx_hbm, table_hbm, out_hbm, tab_v, idx_v0, idx_v1,
                   out_v0, out_v1, in_sems, out_sems):
    idx_bufs = (idx_v0, idx_v1)
    out_bufs = (out_v0, out_v1)
    wid = lax.axis_index("s") * 2 + lax.axis_index("c")
    base = wid * _ROWS_W

    pltpu.sync_copy(table_hbm, tab_v)

    def _exp_body(i):
        sl = pl.ds(i * _LANES, _LANES)
        tab_v[sl] = jnp.exp(tab_v[sl])

    pl.loop(0, _KMERS // _LANES)(_exp_body)

    def _in_slice(c):
        return idx_hbm.at[pl.ds(base + c * _CROWS, _CROWS)]

    def _out_slice(c):
        return out_hbm.at[pl.ds(base + c * _CROWS, _CROWS)]

    for b in range(_NBUF):
        pltpu.async_copy(_in_slice(b), idx_bufs[b], in_sems.at[b])

    for c in range(_NCHUNK):
        b = c % _NBUF
        idx_b = idx_bufs[b]
        out_b = out_bufs[b]
        pltpu.make_async_copy(_in_slice(c), idx_b, in_sems.at[b]).wait()
        if c >= _NBUF:
            pltpu.make_async_copy(out_b, _out_slice(c - _NBUF),
                                  out_sems.at[b]).wait()

        def _row_body(r):
            for g in range(_NGROUPS):
                col = min(g * _LANES, _SEQ - _LANES)
                sl = pl.ds(col, _LANES)
                out_b[r, sl] = plsc.load_gather(tab_v, [idx_b[r, sl]])

        plsc.parallel_loop(0, _CROWS)(_row_body)

        pltpu.async_copy(out_b, _out_slice(c), out_sems.at[b])
        if c + _NBUF < _NCHUNK:
            pltpu.async_copy(_in_slice(c + _NBUF), idx_b, in_sems.at[b])

    for c in range(_NCHUNK - _NBUF, _NCHUNK):
        b = c % _NBUF
        pltpu.make_async_copy(out_bufs[b], _out_slice(c),
                              out_sems.at[b]).wait()


@jax.jit
def kernel(encoded_parents, masks, kmer_embedding):
    del masks
    table_flat = kmer_embedding.reshape(_KMERS)

    mesh = plsc.VectorSubcoreMesh(core_axis_name="c", subcore_axis_name="s")
    out = pl.kernel(
        _gather_kernel,
        mesh=mesh,
        out_type=jax.ShapeDtypeStruct((_BATCH, _SEQ), jnp.float32),
        scratch_types=[
            pltpu.VMEM((_KMERS,), jnp.float32),
            pltpu.VMEM((_CROWS, _SEQ), jnp.int32),
            pltpu.VMEM((_CROWS, _SEQ), jnp.int32),
            pltpu.VMEM((_CROWS, _SEQ), jnp.float32),
            pltpu.VMEM((_CROWS, _SEQ), jnp.float32),
            pltpu.SemaphoreType.DMA((_NBUF,)),
            pltpu.SemaphoreType.DMA((_NBUF,)),
        ],
        compiler_params=pltpu.CompilerParams(needs_layout_passes=False,
                                             use_tc_tiling_on_sc=True),
    )(encoded_parents, table_flat)
    return out


# speedup vs baseline: 770.9677x; 1.8738x over previous
import jax
import jax.numpy as jnp
from jax import lax
from jax.experimental import pallas as pl
from jax.experimental.pallas import tpu as pltpu
from jax.experimental.pallas import tpu_sc as plsc

_BATCH = 16384
_SEQ = 200
_KMERS = 1024
_NW = 32
_UR = 8
_UC = 2048
_NCOLB = _BATCH // _UC
_UNITS = (_SEQ // _UR) * _NCOLB
_KMAIN = _UNITS // _NW
_NEXTRA = _UNITS - _KMAIN * _NW
_LANES = 16


def _gather_kernel(idx_hbm, table_hbm, out_hbm, tab_v, idx_v, out_v,
                   in_sems, out_sems):
    wid = lax.axis_index("s") * 2 + lax.axis_index("c")
    nk = jnp.where(wid < _NEXTRA, _KMAIN + 1, _KMAIN)

    def _unit_slices(u):
        r0 = (u // _NCOLB) * _UR
        c0 = (u % _NCOLB) * _UC
        return pl.ds(r0, _UR), pl.ds(c0, _UC)

    def _start_in(k, off, sem):
        rs, cs = _unit_slices(wid + k * _NW)
        pltpu.async_copy(idx_hbm.at[rs, cs], idx_v.at[pl.ds(off, _UR)], sem)

    _start_in(0, 0, in_sems.at[0])
    _start_in(1, _UR, in_sems.at[1])

    pltpu.sync_copy(table_hbm, tab_v)

    def _exp_body(i):
        sl = pl.ds(i * _LANES, _LANES)
        tab_v[sl] = jnp.exp(tab_v[sl])

    pl.loop(0, _KMERS // _LANES)(_exp_body)

    def _round(k):
        b = k % 2
        off = b * _UR
        rs, cs = _unit_slices(wid + k * _NW)
        pltpu.make_async_copy(idx_hbm.at[rs, cs],
                              idx_v.at[pl.ds(off, _UR)],
                              in_sems.at[b]).wait()

        @pl.when(k >= 2)
        def _wait_prev_store():
            rp, cp = _unit_slices(wid + (k - 2) * _NW)
            pltpu.make_async_copy(out_v.at[pl.ds(off, _UR)],
                                  out_hbm.at[rp, cp], out_sems.at[b]).wait()

        def _row_body(r):
            def _group_body(g):
                sl = pl.ds(g * _LANES, _LANES)
                out_v[off + r, sl] = plsc.load_gather(tab_v,
                                                      [idx_v[off + r, sl]])
            plsc.parallel_loop(0, _UC // _LANES, unroll=8)(_group_body)

        pl.loop(0, _UR)(_row_body)

        pltpu.async_copy(out_v.at[pl.ds(off, _UR)], out_hbm.at[rs, cs],
                         out_sems.at[b])

        @pl.when(k + 2 < nk)
        def _start_next():
            _start_in(k + 2, off, in_sems.at[b])

    pl.loop(0, nk)(_round)

    for b in range(2):
        rs, cs = _unit_slices(wid + b * _NW)
        pltpu.make_async_copy(out_v.at[pl.ds(b * _UR, _UR)],
                              out_hbm.at[rs, cs], out_sems.at[b]).wait()


@jax.jit
def kernel(encoded_parents, masks, kmer_embedding):
    del masks
    idx_t = encoded_parents.T
    table_flat = kmer_embedding.reshape(_KMERS)

    mesh = plsc.VectorSubcoreMesh(core_axis_name="c", subcore_axis_name="s")
    out_t = pl.kernel(
        _gather_kernel,
        mesh=mesh,
        out_type=jax.ShapeDtypeStruct((_SEQ, _BATCH), jnp.float32),
        scratch_types=[
            pltpu.VMEM((_KMERS,), jnp.float32),
            pltpu.VMEM((2 * _UR, _UC), jnp.int32),
            pltpu.VMEM((2 * _UR, _UC), jnp.float32),
            pltpu.SemaphoreType.DMA((2,)),
            pltpu.SemaphoreType.DMA((2,)),
        ],
        compiler_params=pltpu.CompilerParams(needs_layout_passes=False,
                                             use_tc_tiling_on_sc=True),
    )(idx_t, table_flat)
    return out_t.T
